# baseline (speedup 1.0000x reference)
import functools

import jax
import jax.numpy as jnp
from jax import lax
from jax.experimental import pallas as pl
from jax.experimental.pallas import tpu as pltpu
from jax.experimental.pallas import tpu_sc as plsc

B = 16384
F = 26
V = 100000
E = 16
HID = 200
BF = B * F
PLANES = F * E
NW = 32
PPW = PLANES // NW
HALF = B // 2
QCH = B // 4
GW = 128
BBLK = 512


def _sc_emb_gather(tflat, idx_t):
    mesh = plsc.VectorSubcoreMesh(core_axis_name="c", subcore_axis_name="s")

    @functools.partial(
        pl.kernel,
        out_type=jax.ShapeDtypeStruct((PLANES, B), jnp.float32),
        mesh=mesh,
        scratch_types=[
            pltpu.VMEM((V,), jnp.float32),
            pltpu.VMEM((B,), jnp.int32),
            pltpu.VMEM((QCH,), jnp.float32),
            pltpu.VMEM((QCH,), jnp.float32),
            pltpu.SemaphoreType.DMA,
        ],
        compiler_params=pltpu.CompilerParams(needs_layout_passes=False),
    )
    def sc_kernel(tflat_hbm, idx_hbm, out_hbm, row_v, idx_v, val0_v, val1_v,
                  sem):
        wid = lax.axis_index("s") * 2 + lax.axis_index("c")
        base = wid * PPW

        @pl.loop(0, PPW)
        def _plane(p):
            r = base + p
            f = r // E

            @pl.when(jnp.logical_or(p == 0, r % E == 0))
            def _loadidx():
                pltpu.sync_copy(idx_hbm.at[f], idx_v)

            pltpu.sync_copy(tflat_hbm.at[r], row_v)

            for q in range(4):
                val_v = val0_v if q % 2 == 0 else val1_v
                out_slc = out_hbm.at[r, pl.ds(q * QCH, QCH)]

                @pl.when(p * 4 + q >= 2)
                def _drain():
                    pltpu.make_async_copy(val_v, out_slc, sem).wait()

                @pl.loop(0, QCH, step=64)
                def _chunk(i):
                    for u in range(4):
                        iv = idx_v[pl.ds(q * QCH + i + u * 16, 16)]
                        val_v[pl.ds(i + u * 16, 16)] = plsc.load_gather(
                            row_v, [iv])

                pltpu.async_copy(val_v, out_slc, sem)

        last = base + PPW - 1
        pltpu.make_async_copy(
            val0_v, out_hbm.at[last, pl.ds(2 * QCH, QCH)], sem).wait()
        pltpu.make_async_copy(
            val1_v, out_hbm.at[last, pl.ds(3 * QCH, QCH)], sem).wait()

    return sc_kernel(tflat, idx_t)


def _sc_fm_gather(fm_w, idx_t):
    mesh = plsc.VectorSubcoreMesh(core_axis_name="c", subcore_axis_name="s")

    @functools.partial(
        pl.kernel,
        out_type=jax.ShapeDtypeStruct((F, B), jnp.float32),
        mesh=mesh,
        scratch_types=[
            pltpu.VMEM((V,), jnp.float32),
            pltpu.VMEM((HALF,), jnp.int32),
            pltpu.VMEM((HALF,), jnp.float32),
        ],
        compiler_params=pltpu.CompilerParams(
            use_tc_tiling_on_sc=False, needs_layout_passes=False),
    )
    def sc_kernel(fmw_hbm, idx_hbm, out_hbm, row_v, idx_v, val_v):
        wid = lax.axis_index("s") * 2 + lax.axis_index("c")

        @pl.when(wid < F)
        def _field():
            f = wid
            pltpu.sync_copy(fmw_hbm.at[f], row_v)

            @pl.loop(0, 2)
            def _half(h):
                pltpu.sync_copy(idx_hbm.at[f, pl.ds(h * HALF, HALF)], idx_v)

                @pl.loop(0, HALF, step=16)
                def _chunk(i):
                    iv = idx_v[pl.ds(i, 16)]
                    val_v[pl.ds(i, 16)] = plsc.load_gather(row_v, [iv])

                pltpu.sync_copy(val_v, out_hbm.at[f, pl.ds(h * HALF, HALF)])

    return sc_kernel(fm_w.reshape(F, V), idx_t)


def _tc_body(emb_ref, val_ref, fmv_ref, w1_ref, b1_ref, w2_ref, b2_ref,
             w3_ref, b3_ref, wd_ref, bd_ref, out_ref):
    x = emb_ref[...]
    vt = val_ref[...]
    ft = fmv_ref[...]

    first = jnp.sum(ft * vt, axis=0, keepdims=True)

    r_row = lax.broadcasted_iota(jnp.int32, (PLANES, F), 0)
    r_col = lax.broadcasted_iota(jnp.int32, (PLANES, F), 1)
    rmat = (r_row // E == r_col).astype(jnp.float32)
    vexp = jnp.dot(rmat, vt, preferred_element_type=jnp.float32)
    scaled = x * vexp

    s_row = lax.broadcasted_iota(jnp.int32, (E, PLANES), 0)
    s_col = lax.broadcasted_iota(jnp.int32, (E, PLANES), 1)
    smat = (s_col % E == s_row).astype(jnp.float32)
    ssum = jnp.dot(smat, scaled, preferred_element_type=jnp.float32)
    second = 0.5 * (jnp.sum(ssum * ssum, axis=0, keepdims=True)
                    - jnp.sum(scaled * scaled, axis=0, keepdims=True))

    h = jnp.maximum(jnp.dot(w1_ref[...], x,
                            preferred_element_type=jnp.float32) + b1_ref[...], 0.0)
    h = jnp.maximum(jnp.dot(w2_ref[...], h,
                            preferred_element_type=jnp.float32) + b2_ref[...], 0.0)
    h = jnp.maximum(jnp.dot(w3_ref[...], h,
                            preferred_element_type=jnp.float32) + b3_ref[...], 0.0)
    deep = jnp.dot(wd_ref[...], h,
                   preferred_element_type=jnp.float32) + bd_ref[...]

    out_ref[...] = jax.nn.sigmoid(first + second + deep)


def _tc_deepfm(emb_t, val_t, fmv_t, W1t, b1, W2t, b2, W3t, b3, Wdt, bd,
               interpret=False):
    nblk = B // BBLK
    col = lambda i: (0, i)
    rep = lambda i: (0, 0)
    return pl.pallas_call(
        _tc_body,
        grid=(nblk,),
        in_specs=[
            pl.BlockSpec((PLANES, BBLK), col),
            pl.BlockSpec((F, BBLK), col),
            pl.BlockSpec((F, BBLK), col),
            pl.BlockSpec((HID, PLANES), rep),
            pl.BlockSpec((HID, 1), rep),
            pl.BlockSpec((HID, HID), rep),
            pl.BlockSpec((HID, 1), rep),
            pl.BlockSpec((HID, HID), rep),
            pl.BlockSpec((HID, 1), rep),
            pl.BlockSpec((1, HID), rep),
            pl.BlockSpec((1, 1), rep),
        ],
        out_specs=pl.BlockSpec((1, BBLK), col),
        out_shape=jax.ShapeDtypeStruct((1, B), jnp.float32),
        interpret=interpret,
    )(emb_t, val_t, fmv_t, W1t, b1.reshape(HID, 1), W2t, b2.reshape(HID, 1),
      W3t, b3.reshape(HID, 1), Wdt, bd.reshape(1, 1))


def kernel(inputs_index, inputs_value, embed_tables, fm_w,
           W1, b1, W2, b2, W3, b3, Wd, bd):
    tflat = embed_tables.transpose((0, 2, 1)).reshape(PLANES, V)
    idx_t = inputs_index.T.astype(jnp.int32)
    emb_t = _sc_emb_gather(tflat, idx_t)

    idx_t_fm, _ = lax.optimization_barrier((idx_t, emb_t))
    fmv_t = _sc_fm_gather(fm_w, idx_t_fm)

    out_row = _tc_deepfm(emb_t, inputs_value.T, fmv_t,
                         W1.T, b1, W2.T, b2, W3.T, b3, Wd.T, bd)
    return out_row.reshape(B, 1)


# speedup vs baseline: 1.1350x; 1.1350x over previous
import functools

import jax
import jax.numpy as jnp
from jax import lax
from jax.experimental import pallas as pl
from jax.experimental.pallas import tpu as pltpu
from jax.experimental.pallas import tpu_sc as plsc

B = 16384
F = 26
V = 100000
E = 16
HID = 200
BF = B * F
PLANES = F * E
NW = 32
PPW = PLANES // NW
HALF = B // 2
QCH = B // 4
GW = 128
BBLK = 512


def _sc_emb_gather(tflat, idx_t):
    mesh = plsc.VectorSubcoreMesh(core_axis_name="c", subcore_axis_name="s")

    @functools.partial(
        pl.kernel,
        out_type=jax.ShapeDtypeStruct((PLANES, B), jnp.float32),
        mesh=mesh,
        scratch_types=[
            pltpu.VMEM((V,), jnp.float32),
            pltpu.VMEM((HALF,), jnp.int32),
            pltpu.VMEM((HALF,), jnp.float32),
            pltpu.VMEM((HALF,), jnp.float32),
            pltpu.SemaphoreType.DMA,
            pltpu.SemaphoreType.DMA,
        ],
        compiler_params=pltpu.CompilerParams(needs_layout_passes=False),
    )
    def sc_kernel(tflat_hbm, idx_hbm, out_hbm, row_v, idx_v, val0_v, val1_v,
                  sem, isem):
        wid = lax.axis_index("s") * 2 + lax.axis_index("c")
        base = wid * PPW

        def gather_half(val_v):
            @pl.loop(0, HALF, step=64)
            def _chunk(i):
                for u in range(4):
                    iv = idx_v[pl.ds(i + u * 16, 16)]
                    val_v[pl.ds(i + u * 16, 16)] = plsc.load_gather(row_v, [iv])

        @pl.loop(0, PPW)
        def _plane(p):
            r = base + p
            f = r // E
            ih = pltpu.async_copy(idx_hbm.at[f, pl.ds(0, HALF)], idx_v, isem)
            pltpu.sync_copy(tflat_hbm.at[r], row_v)
            ih.wait()

            @pl.when(p > 0)
            def _drain():
                pltpu.make_async_copy(
                    val0_v, out_hbm.at[r - 1, pl.ds(0, HALF)], sem).wait()
                pltpu.make_async_copy(
                    val1_v, out_hbm.at[r - 1, pl.ds(HALF, HALF)], sem).wait()

            gather_half(val0_v)
            pltpu.async_copy(val0_v, out_hbm.at[r, pl.ds(0, HALF)], sem)
            pltpu.sync_copy(idx_hbm.at[f, pl.ds(HALF, HALF)], idx_v)
            gather_half(val1_v)
            pltpu.async_copy(val1_v, out_hbm.at[r, pl.ds(HALF, HALF)], sem)

        last = base + PPW - 1
        pltpu.make_async_copy(
            val0_v, out_hbm.at[last, pl.ds(0, HALF)], sem).wait()
        pltpu.make_async_copy(
            val1_v, out_hbm.at[last, pl.ds(HALF, HALF)], sem).wait()

    return sc_kernel(tflat, idx_t)


def _sc_fm_gather(fm_w, idx_t):
    mesh = plsc.VectorSubcoreMesh(core_axis_name="c", subcore_axis_name="s")

    @functools.partial(
        pl.kernel,
        out_type=jax.ShapeDtypeStruct((F, B), jnp.float32),
        mesh=mesh,
        scratch_types=[
            pltpu.VMEM((V,), jnp.float32),
            pltpu.VMEM((HALF,), jnp.int32),
            pltpu.VMEM((HALF,), jnp.float32),
        ],
        compiler_params=pltpu.CompilerParams(
            use_tc_tiling_on_sc=False, needs_layout_passes=False),
    )
    def sc_kernel(fmw_hbm, idx_hbm, out_hbm, row_v, idx_v, val_v):
        wid = lax.axis_index("s") * 2 + lax.axis_index("c")

        @pl.when(wid < F)
        def _field():
            f = wid
            pltpu.sync_copy(fmw_hbm.at[f], row_v)

            @pl.loop(0, 2)
            def _half(h):
                pltpu.sync_copy(idx_hbm.at[f, pl.ds(h * HALF, HALF)], idx_v)

                @pl.loop(0, HALF, step=16)
                def _chunk(i):
                    iv = idx_v[pl.ds(i, 16)]
                    val_v[pl.ds(i, 16)] = plsc.load_gather(row_v, [iv])

                pltpu.sync_copy(val_v, out_hbm.at[f, pl.ds(h * HALF, HALF)])

    return sc_kernel(fm_w.reshape(F, V), idx_t)


def _tc_body(emb_ref, val_ref, fmv_ref, w1_ref, b1_ref, w2_ref, b2_ref,
             w3_ref, b3_ref, wd_ref, bd_ref, out_ref):
    x = emb_ref[...]
    vt = val_ref[...]
    ft = fmv_ref[...]

    first = jnp.sum(ft * vt, axis=0, keepdims=True)

    r_row = lax.broadcasted_iota(jnp.int32, (PLANES, F), 0)
    r_col = lax.broadcasted_iota(jnp.int32, (PLANES, F), 1)
    rmat = (r_row // E == r_col).astype(jnp.float32)
    vexp = jnp.dot(rmat, vt, preferred_element_type=jnp.float32)
    scaled = x * vexp

    s_row = lax.broadcasted_iota(jnp.int32, (E, PLANES), 0)
    s_col = lax.broadcasted_iota(jnp.int32, (E, PLANES), 1)
    smat = (s_col % E == s_row).astype(jnp.float32)
    ssum = jnp.dot(smat, scaled, preferred_element_type=jnp.float32)
    second = 0.5 * (jnp.sum(ssum * ssum, axis=0, keepdims=True)
                    - jnp.sum(scaled * scaled, axis=0, keepdims=True))

    h = jnp.maximum(jnp.dot(w1_ref[...], x,
                            preferred_element_type=jnp.float32) + b1_ref[...], 0.0)
    h = jnp.maximum(jnp.dot(w2_ref[...], h,
                            preferred_element_type=jnp.float32) + b2_ref[...], 0.0)
    h = jnp.maximum(jnp.dot(w3_ref[...], h,
                            preferred_element_type=jnp.float32) + b3_ref[...], 0.0)
    deep = jnp.dot(wd_ref[...], h,
                   preferred_element_type=jnp.float32) + bd_ref[...]

    out_ref[...] = jax.nn.sigmoid(first + second + deep)


def _tc_deepfm(emb_t, val_t, fmv_t, W1t, b1, W2t, b2, W3t, b3, Wdt, bd,
               interpret=False):
    nblk = B // BBLK
    col = lambda i: (0, i)
    rep = lambda i: (0, 0)
    return pl.pallas_call(
        _tc_body,
        grid=(nblk,),
        in_specs=[
            pl.BlockSpec((PLANES, BBLK), col),
            pl.BlockSpec((F, BBLK), col),
            pl.BlockSpec((F, BBLK), col),
            pl.BlockSpec((HID, PLANES), rep),
            pl.BlockSpec((HID, 1), rep),
            pl.BlockSpec((HID, HID), rep),
            pl.BlockSpec((HID, 1), rep),
            pl.BlockSpec((HID, HID), rep),
            pl.BlockSpec((HID, 1), rep),
            pl.BlockSpec((1, HID), rep),
            pl.BlockSpec((1, 1), rep),
        ],
        out_specs=pl.BlockSpec((1, BBLK), col),
        out_shape=jax.ShapeDtypeStruct((1, B), jnp.float32),
        interpret=interpret,
    )(emb_t, val_t, fmv_t, W1t, b1.reshape(HID, 1), W2t, b2.reshape(HID, 1),
      W3t, b3.reshape(HID, 1), Wdt, bd.reshape(1, 1))


def kernel(inputs_index, inputs_value, embed_tables, fm_w,
           W1, b1, W2, b2, W3, b3, Wd, bd):
    tflat = embed_tables.transpose((0, 2, 1)).reshape(PLANES, V)
    idx_t = inputs_index.T.astype(jnp.int32)
    emb_t = _sc_emb_gather(tflat, idx_t)

    idx_t_fm, _ = lax.optimization_barrier((idx_t, emb_t))
    fmv_t = _sc_fm_gather(fm_w, idx_t_fm)

    out_row = _tc_deepfm(emb_t, inputs_value.T, fmv_t,
                         W1.T, b1, W2.T, b2, W3.T, b3, Wd.T, bd)
    return out_row.reshape(B, 1)


# speedup vs baseline: 1.2131x; 1.0689x over previous
import functools

import jax
import jax.numpy as jnp
from jax import lax
from jax.experimental import pallas as pl
from jax.experimental.pallas import tpu as pltpu
from jax.experimental.pallas import tpu_sc as plsc

B = 16384
F = 26
V = 100000
E = 16
HID = 200
BF = B * F
PLANES = F * E
NW = 32
PPW = PLANES // NW
HALF = B // 2
QCH = B // 4
GW = 128
BBLK = 1024


def _sc_emb_gather(tflat, idx_t):
    mesh = plsc.VectorSubcoreMesh(core_axis_name="c", subcore_axis_name="s")

    @functools.partial(
        pl.kernel,
        out_type=jax.ShapeDtypeStruct((PLANES, B), jnp.float32),
        mesh=mesh,
        scratch_types=[
            pltpu.VMEM((V,), jnp.float32),
            pltpu.VMEM((HALF,), jnp.int32),
            pltpu.VMEM((HALF,), jnp.float32),
            pltpu.VMEM((HALF,), jnp.float32),
            pltpu.SemaphoreType.DMA,
            pltpu.SemaphoreType.DMA,
        ],
        compiler_params=pltpu.CompilerParams(needs_layout_passes=False),
    )
    def sc_kernel(tflat_hbm, idx_hbm, out_hbm, row_v, idx_v, val0_v, val1_v,
                  sem, isem):
        wid = lax.axis_index("s") * 2 + lax.axis_index("c")
        base = wid * PPW

        def gather_half(val_v):
            @pl.loop(0, HALF, step=64)
            def _chunk(i):
                for u in range(4):
                    iv = idx_v[pl.ds(i + u * 16, 16)]
                    val_v[pl.ds(i + u * 16, 16)] = plsc.load_gather(row_v, [iv])

        @pl.loop(0, PPW)
        def _plane(p):
            r = base + p
            f = r // E
            ih = pltpu.async_copy(idx_hbm.at[f, pl.ds(0, HALF)], idx_v, isem)
            pltpu.sync_copy(tflat_hbm.at[r], row_v)
            ih.wait()

            @pl.when(p > 0)
            def _drain():
                pltpu.make_async_copy(
                    val0_v, out_hbm.at[r - 1, pl.ds(0, HALF)], sem).wait()
                pltpu.make_async_copy(
                    val1_v, out_hbm.at[r - 1, pl.ds(HALF, HALF)], sem).wait()

            gather_half(val0_v)
            pltpu.async_copy(val0_v, out_hbm.at[r, pl.ds(0, HALF)], sem)
            pltpu.sync_copy(idx_hbm.at[f, pl.ds(HALF, HALF)], idx_v)
            gather_half(val1_v)
            pltpu.async_copy(val1_v, out_hbm.at[r, pl.ds(HALF, HALF)], sem)

        last = base + PPW - 1
        pltpu.make_async_copy(
            val0_v, out_hbm.at[last, pl.ds(0, HALF)], sem).wait()
        pltpu.make_async_copy(
            val1_v, out_hbm.at[last, pl.ds(HALF, HALF)], sem).wait()

    return sc_kernel(tflat, idx_t)


def _sc_fm_gather(fm_w, idx_t):
    mesh = plsc.VectorSubcoreMesh(core_axis_name="c", subcore_axis_name="s")

    @functools.partial(
        pl.kernel,
        out_type=jax.ShapeDtypeStruct((F, B), jnp.float32),
        mesh=mesh,
        scratch_types=[
            pltpu.VMEM((V,), jnp.float32),
            pltpu.VMEM((HALF,), jnp.int32),
            pltpu.VMEM((HALF,), jnp.float32),
        ],
        compiler_params=pltpu.CompilerParams(
            use_tc_tiling_on_sc=False, needs_layout_passes=False),
    )
    def sc_kernel(fmw_hbm, idx_hbm, out_hbm, row_v, idx_v, val_v):
        wid = lax.axis_index("s") * 2 + lax.axis_index("c")

        @pl.when(wid < F)
        def _field():
            f = wid
            pltpu.sync_copy(fmw_hbm.at[f], row_v)

            @pl.loop(0, 2)
            def _half(h):
                pltpu.sync_copy(idx_hbm.at[f, pl.ds(h * HALF, HALF)], idx_v)

                @pl.loop(0, HALF, step=16)
                def _chunk(i):
                    iv = idx_v[pl.ds(i, 16)]
                    val_v[pl.ds(i, 16)] = plsc.load_gather(row_v, [iv])

                pltpu.sync_copy(val_v, out_hbm.at[f, pl.ds(h * HALF, HALF)])

    return sc_kernel(fm_w.reshape(F, V), idx_t)


def _tc_body(emb_ref, val_ref, fmv_ref, w1_ref, b1_ref, w2_ref, b2_ref,
             w3_ref, b3_ref, wd_ref, bd_ref, out_ref):
    x = emb_ref[...]
    vt = val_ref[...]
    ft = fmv_ref[...]

    first = jnp.sum(ft * vt, axis=0, keepdims=True)

    r_row = lax.broadcasted_iota(jnp.int32, (PLANES, F), 0)
    r_col = lax.broadcasted_iota(jnp.int32, (PLANES, F), 1)
    rmat = (r_row // E == r_col).astype(jnp.float32)
    vexp = jnp.dot(rmat, vt, preferred_element_type=jnp.float32)
    scaled = x * vexp

    s_row = lax.broadcasted_iota(jnp.int32, (E, PLANES), 0)
    s_col = lax.broadcasted_iota(jnp.int32, (E, PLANES), 1)
    smat = (s_col % E == s_row).astype(jnp.float32)
    ssum = jnp.dot(smat, scaled, preferred_element_type=jnp.float32)
    second = 0.5 * (jnp.sum(ssum * ssum, axis=0, keepdims=True)
                    - jnp.sum(scaled * scaled, axis=0, keepdims=True))

    h = jnp.maximum(jnp.dot(w1_ref[...], x,
                            preferred_element_type=jnp.float32) + b1_ref[...], 0.0)
    h = jnp.maximum(jnp.dot(w2_ref[...], h,
                            preferred_element_type=jnp.float32) + b2_ref[...], 0.0)
    h = jnp.maximum(jnp.dot(w3_ref[...], h,
                            preferred_element_type=jnp.float32) + b3_ref[...], 0.0)
    deep = jnp.dot(wd_ref[...], h,
                   preferred_element_type=jnp.float32) + bd_ref[...]

    out_ref[...] = jax.nn.sigmoid(first + second + deep)


def _tc_deepfm(emb_t, val_t, fmv_t, W1t, b1, W2t, b2, W3t, b3, Wdt, bd,
               interpret=False):
    nblk = B // BBLK
    col = lambda i: (0, i)
    rep = lambda i: (0, 0)
    return pl.pallas_call(
        _tc_body,
        grid=(nblk,),
        in_specs=[
            pl.BlockSpec((PLANES, BBLK), col),
            pl.BlockSpec((F, BBLK), col),
            pl.BlockSpec((F, BBLK), col),
            pl.BlockSpec((HID, PLANES), rep),
            pl.BlockSpec((HID, 1), rep),
            pl.BlockSpec((HID, HID), rep),
            pl.BlockSpec((HID, 1), rep),
            pl.BlockSpec((HID, HID), rep),
            pl.BlockSpec((HID, 1), rep),
            pl.BlockSpec((1, HID), rep),
            pl.BlockSpec((1, 1), rep),
        ],
        out_specs=pl.BlockSpec((1, BBLK), col),
        out_shape=jax.ShapeDtypeStruct((1, B), jnp.float32),
        interpret=interpret,
    )(emb_t, val_t, fmv_t, W1t, b1.reshape(HID, 1), W2t, b2.reshape(HID, 1),
      W3t, b3.reshape(HID, 1), Wdt, bd.reshape(1, 1))


def kernel(inputs_index, inputs_value, embed_tables, fm_w,
           W1, b1, W2, b2, W3, b3, Wd, bd):
    tflat = embed_tables.transpose((0, 2, 1)).reshape(PLANES, V)
    idx_t = inputs_index.T.astype(jnp.int32)
    emb_t = _sc_emb_gather(tflat, idx_t)

    idx_t_fm, _ = lax.optimization_barrier((idx_t, emb_t))
    fmv_t = _sc_fm_gather(fm_w, idx_t_fm)

    out_row = _tc_deepfm(emb_t, inputs_value.T, fmv_t,
                         W1.T, b1, W2.T, b2, W3.T, b3, Wd.T, bd)
    return out_row.reshape(B, 1)
